# Initial kernel scaffold; baseline (speedup 1.0000x reference)
import functools

import jax
import jax.numpy as jnp
from jax import lax
from jax.experimental import pallas as pl
from jax.experimental.pallas import tpu as pltpu
from jax.experimental.pallas import tpu_sc as plsc

_N = 10000
_E = 320000
_D = 128

_K = 128
_NTILES = 32
_CH_TOTAL = -(-_E // _K)
_CH_PER_TILE = -(-_CH_TOTAL // _NTILES)
_CH_PAD = _CH_PER_TILE * _NTILES
_E_PAD = _CH_PAD * _K
_N_ACC = 10016
_ZR = _N_ACC // 16
_OR = _N // 16

_mesh = plsc.VectorSubcoreMesh(core_axis_name="c", subcore_axis_name="s")


@functools.partial(
    pl.kernel,
    out_type=jax.ShapeDtypeStruct((2, _N, _D), jnp.float32),
    mesh=_mesh,
    scratch_types=[
        pltpu.VMEM((_CH_PER_TILE * _K,), jnp.int32),
        pltpu.VMEM((_CH_PER_TILE, _K), jnp.int32),
        pltpu.VMEM((_K, _D), jnp.float32),
        pltpu.VMEM_SHARED((_N_ACC, _D), jnp.float32),
        pltpu.SemaphoreType.DMA,
    ],
)
def _sc_aggregate(x_hbm, src_hbm, dst_hbm, zeros_hbm, out_hbm,
                  src_v, dst_v, rows_v, acc_s, sem):
    cid = lax.axis_index("c")
    sid = lax.axis_index("s")
    wid = cid * 16 + sid

    pltpu.sync_copy(zeros_hbm, acc_s.at[pl.ds(sid * _ZR, _ZR)])

    t0 = wid * _CH_PER_TILE
    pltpu.sync_copy(src_hbm.at[pl.ds(t0 * _K, _CH_PER_TILE * _K)], src_v)
    pltpu.sync_copy(dst_hbm.at[pl.ds(t0, _CH_PER_TILE)], dst_v)

    plsc.subcore_barrier()

    def body(c, carry):
        pltpu.async_copy(x_hbm.at[src_v.at[pl.ds(c * _K, _K)]],
                         rows_v, sem).wait()
        pltpu.sync_copy(rows_v, acc_s.at[dst_v.at[c]], add=True)
        return carry

    lax.fori_loop(0, _CH_PER_TILE, body, 0)

    plsc.subcore_barrier()

    pltpu.sync_copy(acc_s.at[pl.ds(sid * _OR, _OR)],
                    out_hbm.at[cid, pl.ds(sid * _OR, _OR)])


def _tc_body(p_ref, w_ref, b_ref, o_ref):
    acc = p_ref[0] + p_ref[1]
    h = lax.dot_general(acc, w_ref[...], (((1,), (1,)), ((), ())),
                        preferred_element_type=jnp.float32)
    o_ref[...] = jnp.maximum(h + b_ref[...], 0.0)


_tc_apply = pl.pallas_call(
    _tc_body,
    grid=(10,),
    in_specs=[
        pl.BlockSpec((2, _N // 10, _D), lambda i: (0, i, 0)),
        pl.BlockSpec((_D, _D), lambda i: (0, 0)),
        pl.BlockSpec((1, _D), lambda i: (0, 0)),
    ],
    out_specs=pl.BlockSpec((_N // 10, _D), lambda i: (i, 0)),
    out_shape=jax.ShapeDtypeStruct((_N, _D), jnp.float32),
)


def kernel(x, edge_index, W, b):
    src = edge_index[0]
    dst = edge_index[1]
    pad = _E_PAD - _E
    src_p = jnp.concatenate([src, jnp.zeros((pad,), jnp.int32)])
    dst_p = jnp.concatenate([dst, jnp.full((pad,), _N, jnp.int32)])
    dst2 = dst_p.reshape(_CH_PAD, _K)
    zeros = jnp.zeros((_ZR, _D), jnp.float32)
    partials = _sc_aggregate(x, src_p, dst2, zeros)
    return _tc_apply(partials, W, b.reshape(1, _D))


# speedup vs baseline: 3.0009x; 3.0009x over previous
import functools

import jax
import jax.numpy as jnp
from jax import lax
from jax.experimental import pallas as pl
from jax.experimental.pallas import tpu as pltpu
from jax.experimental.pallas import tpu_sc as plsc

_N = 10000
_E = 320000
_D = 128

_K = 128
_NTILES = 32
_CH_PER_TILE = 80
_CH_PAD = _CH_PER_TILE * _NTILES
_E_PAD = _CH_PAD * _K
_N_ACC = 10112
_ZR = _N_ACC // 16
_OR_LAST = _N - 15 * _ZR

_mesh = plsc.VectorSubcoreMesh(core_axis_name="c", subcore_axis_name="s")


@functools.partial(
    pl.kernel,
    out_type=jax.ShapeDtypeStruct((2, _N, _D), jnp.float32),
    mesh=_mesh,
    scratch_types=[
        pltpu.VMEM((_CH_PER_TILE * _K,), jnp.int32),
        pltpu.VMEM((_CH_PER_TILE, _K), jnp.int32),
        pltpu.VMEM((_K, _D), jnp.float32),
        pltpu.VMEM_SHARED((_N_ACC, _D), jnp.float32),
        pltpu.SemaphoreType.DMA,
    ],
)
def _sc_aggregate(x_hbm, src_hbm, dst_hbm, zeros_hbm, out_hbm,
                  src_v, dst_v, rows_v, acc_s, sem):
    cid = lax.axis_index("c")
    sid = lax.axis_index("s")
    wid = cid * 16 + sid

    pltpu.sync_copy(zeros_hbm, acc_s.at[pl.ds(sid * _ZR, _ZR)])

    t0 = wid * _CH_PER_TILE
    pltpu.sync_copy(src_hbm.at[pl.ds(t0 * _K, _CH_PER_TILE * _K)], src_v)
    pltpu.sync_copy(dst_hbm.at[pl.ds(t0, _CH_PER_TILE)], dst_v)

    plsc.subcore_barrier()

    def body(c, carry):
        pltpu.async_copy(x_hbm.at[src_v.at[pl.ds(c * _K, _K)]],
                         rows_v, sem).wait()
        pltpu.sync_copy(rows_v, acc_s.at[dst_v.at[c]], add=True)
        return carry

    lax.fori_loop(0, _CH_PER_TILE, body, 0)

    plsc.subcore_barrier()

    @pl.when(sid < 15)
    def _():
        pltpu.sync_copy(acc_s.at[pl.ds(sid * _ZR, _ZR)],
                        out_hbm.at[cid, pl.ds(sid * _ZR, _ZR)])

    @pl.when(sid == 15)
    def _():
        pltpu.sync_copy(acc_s.at[pl.ds(15 * _ZR, _OR_LAST)],
                        out_hbm.at[cid, pl.ds(15 * _ZR, _OR_LAST)])


def _tc_body(p_ref, w_ref, b_ref, o_ref):
    acc = p_ref[0] + p_ref[1]
    h = lax.dot_general(acc, w_ref[...], (((1,), (1,)), ((), ())),
                        preferred_element_type=jnp.float32)
    o_ref[...] = jnp.maximum(h + b_ref[...], 0.0)


_tc_apply = pl.pallas_call(
    _tc_body,
    grid=(10,),
    in_specs=[
        pl.BlockSpec((2, _N // 10, _D), lambda i: (0, i, 0)),
        pl.BlockSpec((_D, _D), lambda i: (0, 0)),
        pl.BlockSpec((1, _D), lambda i: (0, 0)),
    ],
    out_specs=pl.BlockSpec((_N // 10, _D), lambda i: (i, 0)),
    out_shape=jax.ShapeDtypeStruct((_N, _D), jnp.float32),
)


def kernel(x, edge_index, W, b):
    src = edge_index[0]
    dst = edge_index[1]
    pad = _E_PAD - _E
    src_p = jnp.concatenate([src, jnp.zeros((pad,), jnp.int32)])
    dst_p = jnp.concatenate([dst, jnp.full((pad,), _N, jnp.int32)])
    dst2 = dst_p.reshape(_CH_PAD, _K)
    zeros = jnp.zeros((_ZR, _D), jnp.float32)
    partials = _sc_aggregate(x, src_p, dst2, zeros)
    return _tc_apply(partials, W, b.reshape(1, _D))


# speedup vs baseline: 3.3511x; 1.1167x over previous
import functools

import jax
import jax.numpy as jnp
from jax import lax
from jax.experimental import pallas as pl
from jax.experimental.pallas import tpu as pltpu
from jax.experimental.pallas import tpu_sc as plsc

_N = 10000
_E = 320000
_D = 128

_K = 128
_NTILES = 32
_CH_PER_TILE = 80
_CH_PAD = _CH_PER_TILE * _NTILES
_E_PAD = _CH_PAD * _K
_N_ACC = 10112
_ZR = _N_ACC // 16
_OR_LAST = _N - 15 * _ZR
_HALF = _CH_PER_TILE // 2

_mesh = plsc.VectorSubcoreMesh(core_axis_name="c", subcore_axis_name="s")


@functools.partial(
    pl.kernel,
    out_type=jax.ShapeDtypeStruct((2, _N, _D), jnp.float32),
    mesh=_mesh,
    scratch_types=[
        pltpu.VMEM((_HALF * _K,), jnp.int32),
        pltpu.VMEM((_HALF, _K), jnp.int32),
        pltpu.VMEM((_K, _D), jnp.float32),
        pltpu.VMEM((_K, _D), jnp.float32),
        pltpu.VMEM_SHARED((_N_ACC, _D), jnp.float32),
        pltpu.SemaphoreType.DMA,
        pltpu.SemaphoreType.DMA,
    ],
)
def _sc_aggregate(x_hbm, src_hbm, dst_hbm, zeros_hbm, out_hbm,
                  src_v, dst_v, rows_a, rows_b, acc_s, sem_a, sem_b):
    cid = lax.axis_index("c")
    sid = lax.axis_index("s")
    wid = cid * 16 + sid

    pltpu.sync_copy(zeros_hbm, acc_s.at[pl.ds(sid * _ZR, _ZR)])
    plsc.subcore_barrier()

    t0 = wid * _CH_PER_TILE

    def _start_gather(c, buf, sem):
        pltpu.async_copy(x_hbm.at[src_v.at[pl.ds(c * _K, _K)]], buf, sem)

    def _wait_gather(buf, sem):
        pltpu.make_async_copy(x_hbm.at[pl.ds(0, _K)], buf, sem).wait()

    for h in range(_CH_PER_TILE // _HALF):
        base = t0 + h * _HALF
        pltpu.sync_copy(src_hbm.at[pl.ds(base * _K, _HALF * _K)], src_v)
        pltpu.sync_copy(dst_hbm.at[pl.ds(base, _HALF)], dst_v)
        _start_gather(0, rows_a, sem_a)

        def body(i, carry):
            c0 = 2 * i
            _start_gather(c0 + 1, rows_b, sem_b)
            _wait_gather(rows_a, sem_a)
            pltpu.sync_copy(rows_a, acc_s.at[dst_v.at[c0]], add=True)

            @pl.when(c0 + 2 < _HALF)
            def _():
                _start_gather(c0 + 2, rows_a, sem_a)

            _wait_gather(rows_b, sem_b)
            pltpu.sync_copy(rows_b, acc_s.at[dst_v.at[c0 + 1]], add=True)
            return carry

        lax.fori_loop(0, _HALF // 2, body, 0)

    plsc.subcore_barrier()

    @pl.when(sid < 15)
    def _():
        pltpu.sync_copy(acc_s.at[pl.ds(sid * _ZR, _ZR)],
                        out_hbm.at[cid, pl.ds(sid * _ZR, _ZR)])

    @pl.when(sid == 15)
    def _():
        pltpu.sync_copy(acc_s.at[pl.ds(15 * _ZR, _OR_LAST)],
                        out_hbm.at[cid, pl.ds(15 * _ZR, _OR_LAST)])


def _tc_body(p_ref, w_ref, b_ref, o_ref):
    acc = p_ref[0] + p_ref[1]
    h = lax.dot_general(acc, w_ref[...], (((1,), (1,)), ((), ())),
                        preferred_element_type=jnp.float32)
    o_ref[...] = jnp.maximum(h + b_ref[...], 0.0)


_tc_apply = pl.pallas_call(
    _tc_body,
    grid=(10,),
    in_specs=[
        pl.BlockSpec((2, _N // 10, _D), lambda i: (0, i, 0)),
        pl.BlockSpec((_D, _D), lambda i: (0, 0)),
        pl.BlockSpec((1, _D), lambda i: (0, 0)),
    ],
    out_specs=pl.BlockSpec((_N // 10, _D), lambda i: (i, 0)),
    out_shape=jax.ShapeDtypeStruct((_N, _D), jnp.float32),
)


def kernel(x, edge_index, W, b):
    src = edge_index[0]
    dst = edge_index[1]
    pad = _E_PAD - _E
    src_p = jnp.concatenate([src, jnp.zeros((pad,), jnp.int32)])
    dst_p = jnp.concatenate([dst, jnp.full((pad,), _N, jnp.int32)])
    dst2 = dst_p.reshape(_CH_PAD, _K)
    zeros = jnp.zeros((_ZR, _D), jnp.float32)
    partials = _sc_aggregate(x, src_p, dst2, zeros)
    return _tc_apply(partials, W, b.reshape(1, _D))


# speedup vs baseline: 12.3765x; 3.6932x over previous
import functools

import jax
import jax.numpy as jnp
from jax import lax
from jax.experimental import pallas as pl
from jax.experimental.pallas import tpu as pltpu
from jax.experimental.pallas import tpu_sc as plsc

_N = 10000
_E = 320000
_D = 128

_K = 128
_NTILES = 32
_CH_TOTAL = _E // _K
_CH_PER_TILE = 80
_CH_LAST = _CH_TOTAL - 31 * _CH_PER_TILE
_CH_LAST_PAD = 24
_N_ACC = 10112
_ZR = _N_ACC // 16
_OR_LAST = _N - 15 * _ZR
_HALF = _CH_PER_TILE // 2

_mesh = plsc.VectorSubcoreMesh(core_axis_name="c", subcore_axis_name="s")


@functools.partial(
    pl.kernel,
    out_type=jax.ShapeDtypeStruct((2, _N, _D), jnp.float32),
    mesh=_mesh,
    scratch_types=[
        pltpu.VMEM((_HALF * _K,), jnp.int32),
        pltpu.VMEM((_HALF, _K), jnp.int32),
        pltpu.VMEM((_K, _D), jnp.float32),
        pltpu.VMEM((_K, _D), jnp.float32),
        pltpu.VMEM_SHARED((_N_ACC, _D), jnp.float32),
        pltpu.SemaphoreType.DMA,
        pltpu.SemaphoreType.DMA,
    ],
)
def _sc_aggregate(x_hbm, src_hbm, dst_hbm, zeros_hbm, out_hbm,
                  src_v, dst_v, rows_a, rows_b, acc_s, sem_a, sem_b):
    cid = lax.axis_index("c")
    sid = lax.axis_index("s")
    wid = cid * 16 + sid

    pltpu.sync_copy(zeros_hbm, acc_s.at[pl.ds(sid * _ZR, _ZR)])
    plsc.subcore_barrier()

    t0 = wid * _CH_PER_TILE

    def _start_gather(c, buf, sem):
        pltpu.async_copy(x_hbm.at[src_v.at[pl.ds(c * _K, _K)]], buf, sem)

    def _wait_gather(buf, sem):
        pltpu.make_async_copy(x_hbm.at[pl.ds(0, _K)], buf, sem).wait()

    def _stage(base, n):
        pltpu.sync_copy(src_hbm.at[pl.ds(base * _K, n * _K)],
                        src_v.at[pl.ds(0, n * _K)])
        pltpu.sync_copy(dst_hbm.at[pl.ds(base, n)], dst_v.at[pl.ds(0, n)])

    def _run_chunks(n):
        _start_gather(0, rows_a, sem_a)

        def body(i, carry):
            c0 = 2 * i
            _start_gather(c0 + 1, rows_b, sem_b)
            _wait_gather(rows_a, sem_a)
            pltpu.sync_copy(rows_a, acc_s.at[dst_v.at[c0]], add=True)

            @pl.when(c0 + 2 < n)
            def _():
                _start_gather(c0 + 2, rows_a, sem_a)

            _wait_gather(rows_b, sem_b)
            pltpu.sync_copy(rows_b, acc_s.at[dst_v.at[c0 + 1]], add=True)
            return carry

        lax.fori_loop(0, n // 2, body, 0)

    @pl.when(wid < 31)
    def _():
        for h in range(_CH_PER_TILE // _HALF):
            _stage(t0 + h * _HALF, _HALF)
            _run_chunks(_HALF)

    @pl.when(wid == 31)
    def _():
        pltpu.sync_copy(src_hbm.at[pl.ds(t0 * _K, _CH_LAST * _K)],
                        src_v.at[pl.ds(0, _CH_LAST * _K)])
        pltpu.sync_copy(dst_hbm.at[pl.ds(t0, _CH_LAST_PAD)],
                        dst_v.at[pl.ds(0, _CH_LAST_PAD)])
        _run_chunks(_CH_LAST)

    plsc.subcore_barrier()

    @pl.when(sid < 15)
    def _():
        pltpu.sync_copy(acc_s.at[pl.ds(sid * _ZR, _ZR)],
                        out_hbm.at[cid, pl.ds(sid * _ZR, _ZR)])

    @pl.when(sid == 15)
    def _():
        pltpu.sync_copy(acc_s.at[pl.ds(15 * _ZR, _OR_LAST)],
                        out_hbm.at[cid, pl.ds(15 * _ZR, _OR_LAST)])


def _tc_body(p_ref, w_ref, b_ref, o_ref):
    acc = p_ref[0] + p_ref[1]
    h = lax.dot_general(acc, w_ref[...], (((1,), (1,)), ((), ())),
                        preferred_element_type=jnp.float32)
    o_ref[...] = jnp.maximum(h + b_ref[...], 0.0)


_tc_apply = pl.pallas_call(
    _tc_body,
    grid=(10,),
    in_specs=[
        pl.BlockSpec((2, _N // 10, _D), lambda i: (0, i, 0)),
        pl.BlockSpec((_D, _D), lambda i: (0, 0)),
        pl.BlockSpec((1, _D), lambda i: (0, 0)),
    ],
    out_specs=pl.BlockSpec((_N // 10, _D), lambda i: (i, 0)),
    out_shape=jax.ShapeDtypeStruct((_N, _D), jnp.float32),
)


def kernel(x, edge_index, W, b):
    src = edge_index[0]
    pad = (_CH_LAST_PAD - _CH_LAST) * _K
    dst_pad = _N + (jnp.arange(pad, dtype=jnp.int32) % (_N_ACC - _N))
    dst2 = jnp.concatenate([edge_index[1], dst_pad]).reshape(-1, _K)
    zeros = jnp.zeros((_ZR, _D), jnp.float32)
    partials = _sc_aggregate(x, src, dst2, zeros)
    return _tc_apply(partials, W, b.reshape(1, _D))


# speedup vs baseline: 12.8397x; 1.0374x over previous
import functools

import jax
import jax.numpy as jnp
from jax import lax
from jax.experimental import pallas as pl
from jax.experimental.pallas import tpu as pltpu
from jax.experimental.pallas import tpu_sc as plsc

_N = 10000
_E = 320000
_D = 128

_K = 128
_NTILES = 32
_CH_TOTAL = _E // _K
_CH_PER_TILE = 80
_CH_LAST = _CH_TOTAL - 31 * _CH_PER_TILE
_N_ACC = 10112
_ZR = _N_ACC // 16
_OR_LAST = _N - 15 * _ZR

_mesh = plsc.VectorSubcoreMesh(core_axis_name="c", subcore_axis_name="s")


@functools.partial(
    pl.kernel,
    out_type=jax.ShapeDtypeStruct((2, _N, _D), jnp.float32),
    mesh=_mesh,
    scratch_types=[
        pltpu.VMEM((_CH_PER_TILE * _K,), jnp.int32),
        pltpu.VMEM((_K,), jnp.int32),
        pltpu.VMEM((_K,), jnp.int32),
        pltpu.VMEM((_K, _D), jnp.float32),
        pltpu.VMEM((_K, _D), jnp.float32),
        pltpu.VMEM_SHARED((_N_ACC, _D), jnp.float32),
        pltpu.SemaphoreType.DMA,
        pltpu.SemaphoreType.DMA,
        pltpu.SemaphoreType.DMA,
        pltpu.SemaphoreType.DMA,
    ],
)
def _sc_aggregate(x_hbm, src_hbm, dst_hbm, zeros_hbm, out_hbm,
                  src_v, dst_a, dst_b, rows_a, rows_b, acc_s,
                  sem_ga, sem_gb, sem_da, sem_db):
    cid = lax.axis_index("c")
    sid = lax.axis_index("s")
    wid = cid * 16 + sid

    pltpu.sync_copy(zeros_hbm, acc_s.at[pl.ds(sid * _ZR, _ZR)])
    plsc.subcore_barrier()

    t0 = wid * _CH_PER_TILE

    def _start_chunk(c, rbuf, dbuf, gsem, dsem):
        pltpu.async_copy(dst_hbm.at[pl.ds((t0 + c) * _K, _K)], dbuf, dsem)
        pltpu.async_copy(x_hbm.at[src_v.at[pl.ds(c * _K, _K)]], rbuf, gsem)

    def _finish_chunk(rbuf, dbuf, gsem, dsem):
        pltpu.make_async_copy(dst_hbm.at[pl.ds(0, _K)], dbuf, dsem).wait()
        pltpu.make_async_copy(x_hbm.at[pl.ds(0, _K)], rbuf, gsem).wait()
        pltpu.sync_copy(rbuf, acc_s.at[dbuf], add=True)

    def _run_chunks(n):
        _start_chunk(0, rows_a, dst_a, sem_ga, sem_da)

        def body(i, carry):
            c0 = 2 * i
            _start_chunk(c0 + 1, rows_b, dst_b, sem_gb, sem_db)
            _finish_chunk(rows_a, dst_a, sem_ga, sem_da)

            @pl.when(c0 + 2 < n)
            def _():
                _start_chunk(c0 + 2, rows_a, dst_a, sem_ga, sem_da)

            _finish_chunk(rows_b, dst_b, sem_gb, sem_db)
            return carry

        lax.fori_loop(0, n // 2, body, 0)

    @pl.when(wid < 31)
    def _():
        pltpu.sync_copy(src_hbm.at[pl.ds(t0 * _K, _CH_PER_TILE * _K)],
                        src_v)
        _run_chunks(_CH_PER_TILE)

    @pl.when(wid == 31)
    def _():
        pltpu.sync_copy(src_hbm.at[pl.ds(t0 * _K, _CH_LAST * _K)],
                        src_v.at[pl.ds(0, _CH_LAST * _K)])
        _run_chunks(_CH_LAST)

    plsc.subcore_barrier()

    @pl.when(sid < 15)
    def _():
        pltpu.sync_copy(acc_s.at[pl.ds(sid * _ZR, _ZR)],
                        out_hbm.at[cid, pl.ds(sid * _ZR, _ZR)])

    @pl.when(sid == 15)
    def _():
        pltpu.sync_copy(acc_s.at[pl.ds(15 * _ZR, _OR_LAST)],
                        out_hbm.at[cid, pl.ds(15 * _ZR, _OR_LAST)])


def _tc_body(p_ref, w_ref, b_ref, o_ref):
    acc = p_ref[0] + p_ref[1]
    h = lax.dot_general(acc, w_ref[...], (((1,), (1,)), ((), ())),
                        preferred_element_type=jnp.float32)
    o_ref[...] = jnp.maximum(h + b_ref[...], 0.0)


_tc_apply = pl.pallas_call(
    _tc_body,
    grid=(10,),
    in_specs=[
        pl.BlockSpec((2, _N // 10, _D), lambda i: (0, i, 0)),
        pl.BlockSpec((_D, _D), lambda i: (0, 0)),
        pl.BlockSpec((1, _D), lambda i: (0, 0)),
    ],
    out_specs=pl.BlockSpec((_N // 10, _D), lambda i: (i, 0)),
    out_shape=jax.ShapeDtypeStruct((_N, _D), jnp.float32),
)


def kernel(x, edge_index, W, b):
    zeros = jnp.zeros((_ZR, _D), jnp.float32)
    partials = _sc_aggregate(x, edge_index[0], edge_index[1], zeros)
    return _tc_apply(partials, W, b.reshape(1, _D))
